# baseline (speedup 1.0000x reference)
import functools

import jax
import jax.numpy as jnp
from jax import lax
from jax.experimental import pallas as pl
from jax.experimental.pallas import tpu as pltpu
from jax.experimental.pallas import tpu_sc as plsc

N_NODES = 10000
N_EDGES = 320000
D_FEAT = 128
LATENT = 64

NP = 10240
EP = 327680
GW = 128
EBLK = 2048

_EPS = 1e-5


def _collapse(mlp):
    layers = mlp["layers"]
    W, b = layers[0]
    for Wi, bi in layers[1:]:
        W = W @ Wi
        b = b @ Wi + bi
    if mlp["ln"] is not None:
        g, bt = mlp["ln"]
    else:
        g, bt = None, None
    return W, b, g, bt


def _ln(x, g, bt):
    mu = jnp.mean(x, axis=-1, keepdims=True)
    d = x - mu
    var = jnp.mean(d * d, axis=-1, keepdims=True)
    return d * lax.rsqrt(var + _EPS) * g + bt


def _row(c, i):
    return c[i, :][None, :]


def _prep_body(nf_ref, wenc_ref, c_ref, wsr_ref, nlat_ref, t_ref):
    c = c_ref[...]
    x = jnp.dot(nf_ref[...], wenc_ref[...], preferred_element_type=jnp.float32,
                precision=lax.Precision.HIGHEST)
    x = x + _row(c, 0)
    nlat = _ln(x, _row(c, 1), _row(c, 2))
    nlat_ref[...] = nlat
    t_ref[...] = jnp.dot(nlat, wsr_ref[...], preferred_element_type=jnp.float32,
                precision=lax.Precision.HIGHEST)


def _edge1_body(ef_ref, gs_ref, gr_ref, wenc_ref, we1_ref, we2_ref, c_ref,
                p1_ref):
    c = c_ref[...]
    e0 = jnp.dot(ef_ref[...], wenc_ref[...], preferred_element_type=jnp.float32,
                precision=lax.Precision.HIGHEST)
    e0 = _ln(e0 + _row(c, 0), _row(c, 1), _row(c, 2))
    pre = gs_ref[:, 0:LATENT] + gr_ref[:, LATENT:2 * LATENT]
    pre = pre + jnp.dot(e0, we1_ref[...], preferred_element_type=jnp.float32,
                precision=lax.Precision.HIGHEST)
    pre = pre + _row(c, 3)
    ne1 = _ln(pre, _row(c, 4), _row(c, 5))
    el1 = e0 + ne1
    ehat2 = (jnp.dot(el1, we2_ref[...], preferred_element_type=jnp.float32,
                precision=lax.Precision.HIGHEST)
             + _row(c, 6))
    p1_ref[...] = jnp.concatenate([ne1, ehat2], axis=1)


def _edge2_body(p1_ref, gs_ref, gr_ref, c_ref, p2_ref):
    c = c_ref[...]
    pre = (gs_ref[:, 0:LATENT] + gr_ref[:, LATENT:2 * LATENT]
           + p1_ref[:, LATENT:2 * LATENT])
    ne2 = _ln(pre, _row(c, 0), _row(c, 1))
    p2_ref[...] = jnp.concatenate([ne2, jnp.zeros_like(ne2)], axis=1)


def _node_body(nlat_ref, agg_ref, wn1_ref, wn2_ref, c_ref, wsr_ref,
               nlat1_ref, t_ref):
    c = c_ref[...]
    agg = agg_ref[0, :, 0:LATENT] + agg_ref[1, :, 0:LATENT]
    nlat = nlat_ref[...]
    x = jnp.dot(nlat, wn1_ref[...], preferred_element_type=jnp.float32,
                precision=lax.Precision.HIGHEST)
    x = x + jnp.dot(agg, wn2_ref[...], preferred_element_type=jnp.float32,
                precision=lax.Precision.HIGHEST)
    x = x + _row(c, 0)
    nlat1 = nlat + _ln(x, _row(c, 1), _row(c, 2))
    nlat1_ref[...] = nlat1
    t_ref[...] = jnp.dot(nlat1, wsr_ref[...], preferred_element_type=jnp.float32,
                precision=lax.Precision.HIGHEST)


def _node2dec_body(nlat_ref, agg_ref, wn1_ref, wn2_ref, c_ref, wd_ref, bd_ref,
                   out_ref):
    c = c_ref[...]
    agg = agg_ref[0, :, 0:LATENT] + agg_ref[1, :, 0:LATENT]
    nlat = nlat_ref[...]
    x = jnp.dot(nlat, wn1_ref[...], preferred_element_type=jnp.float32,
                precision=lax.Precision.HIGHEST)
    x = x + jnp.dot(agg, wn2_ref[...], preferred_element_type=jnp.float32,
                precision=lax.Precision.HIGHEST)
    x = x + _row(c, 0)
    nlat2 = nlat + _ln(x, _row(c, 1), _row(c, 2))
    out_ref[...] = (
        jnp.dot(nlat2, wd_ref[...], preferred_element_type=jnp.float32,
                precision=lax.Precision.HIGHEST)
        + bd_ref[...])


def _whole(x):
    return pl.BlockSpec(x.shape, lambda *_: (0,) * x.ndim)


def _tc_call(body, outs, *args):
    out_shape = [jax.ShapeDtypeStruct(s, d) for s, d in outs]
    return pl.pallas_call(
        body,
        grid=(1,),
        in_specs=[_whole(a) for a in args],
        out_specs=[pl.BlockSpec(s, lambda *_: (0,) * len(s)) for s, _ in outs],
        out_shape=out_shape,
    )(*args)


def _tc_edge_call(body, n_out, row_args, const_args):
    def rspec(a):
        return pl.BlockSpec((EBLK, a.shape[1]), lambda i: (i, 0))

    in_specs = ([rspec(a) for a in row_args]
                + [_whole(a) for a in const_args])
    out_specs = [pl.BlockSpec((EBLK, 2 * LATENT), lambda i: (i, 0))
                 for _ in range(n_out)]
    out_shape = [jax.ShapeDtypeStruct((EP, 2 * LATENT), jnp.float32)
                 for _ in range(n_out)]
    res = pl.pallas_call(
        body,
        grid=(EP // EBLK,),
        in_specs=in_specs,
        out_specs=out_specs,
        out_shape=out_shape,
    )(*row_args, *const_args)
    return res


def _sc_mesh():
    return plsc.VectorSubcoreMesh(core_axis_name="c", subcore_axis_name="s")


def _sc_gather(tab, s_idx, r_idx):
    GW2 = 2 * GW
    s2 = s_idx.reshape(EP // GW, GW)
    r2 = r_idx.reshape(EP // GW, GW)
    out_type = (jax.ShapeDtypeStruct((EP, 2 * LATENT), jnp.float32),
                jax.ShapeDtypeStruct((EP, 2 * LATENT), jnp.float32))

    @functools.partial(
        pl.kernel, out_type=out_type, mesh=_sc_mesh(),
        scratch_types=[pltpu.SemaphoreType.DMA] * 4)
    def k(t_hbm, s_hbm, r_hbm, os_hbm, or_hbm, s0, s1, s2_, s3):
        def body(s_vmem, r_vmem, os_vmem, or_vmem):
            c0 = pltpu.async_copy(t_hbm.at[s_vmem.at[0]],
                                  os_vmem.at[pl.ds(0, GW)], s0)
            c1 = pltpu.async_copy(t_hbm.at[r_vmem.at[0]],
                                  or_vmem.at[pl.ds(0, GW)], s1)
            c2 = pltpu.async_copy(t_hbm.at[s_vmem.at[1]],
                                  os_vmem.at[pl.ds(GW, GW)], s2_)
            c3 = pltpu.async_copy(t_hbm.at[r_vmem.at[1]],
                                  or_vmem.at[pl.ds(GW, GW)], s3)
            c0.wait()
            c1.wait()
            c2.wait()
            c3.wait()

        obspec = pl.BlockSpec((GW2, 2 * LATENT), lambda i: (i, 0),
                              pipeline_mode=pl.Buffered(buffer_count=1))
        pltpu.emit_pipeline(
            body,
            grid=(EP // GW2,),
            in_specs=[pl.BlockSpec((2, GW), lambda i: (i, 0)),
                      pl.BlockSpec((2, GW), lambda i: (i, 0))],
            out_specs=[obspec, obspec],
            core_axis_name=("c", "s"),
            dimension_semantics=(pltpu.PARALLEL,),
        )(s_hbm, r_hbm, os_hbm, or_hbm)

    return k(tab, s2, r2)


def _sc_scatter(ne, r_idx, zeros_np):
    W = ne.shape[1]
    r2 = r_idx.reshape(1, EP)
    rows = NP // 16
    out_type = jax.ShapeDtypeStruct((2, NP, W), jnp.float32)

    @functools.partial(
        pl.kernel, out_type=out_type, mesh=_sc_mesh(),
        scratch_types=[pltpu.VMEM_SHARED((NP, W), jnp.float32)])
    def k(ne_hbm, r_hbm, z_hbm, out_hbm, acc):
        cid = lax.axis_index("c")
        sid = lax.axis_index("s")
        base = pl.multiple_of(sid * rows, 8)
        pltpu.sync_copy(z_hbm.at[pl.ds(base, rows)],
                        acc.at[pl.ds(base, rows)])
        plsc.subcore_barrier()

        def body(ne_vmem, i_vmem):
            pltpu.sync_copy(ne_vmem, acc.at[i_vmem.at[0]], add=True)

        pltpu.emit_pipeline(
            body,
            grid=(EP // GW,),
            in_specs=[pl.BlockSpec((GW, W), lambda i: (i, 0)),
                      pl.BlockSpec((1, GW), lambda i: (0, i))],
            out_specs=[],
            core_axis_name=("c", "s"),
            dimension_semantics=(pltpu.PARALLEL,),
        )(ne_hbm, r_hbm)

        plsc.subcore_barrier()
        pltpu.sync_copy(acc.at[pl.ds(base, rows)],
                        out_hbm.at[cid, pl.ds(base, rows)])

    return k(ne, r2, zeros_np)


def kernel(node_features, edge_features, senders, receivers, params):
    f32 = jnp.float32

    Wen, ben, gen, bten = _collapse(params["enc_node"])
    Wee, bee, gee, btee = _collapse(params["enc_edge"])
    blocks = []
    for blk in params["blocks"]:
        We, be, ge, bte = _collapse(blk["edge"])
        Wn, bn, gn, btn = _collapse(blk["node"])
        Wsr = jnp.concatenate([We[0:64], We[64:128]], axis=1)
        blocks.append(dict(
            Wsr=Wsr, We=We[128:192], be=be, ge=ge, bte=bte,
            Wn1=Wn[0:64], Wn2=Wn[64:128], bn=bn, gn=gn, btn=btn))
    Wd, bd, _, _ = _collapse(params["dec"])
    Wd_p = jnp.zeros((LATENT, 128), f32).at[:, :3].set(Wd)
    bd_p = jnp.zeros((1, 128), f32).at[0, :3].set(bd)

    nf_p = jnp.zeros((NP, D_FEAT), f32).at[:N_NODES].set(node_features)
    ef_p = jnp.zeros((EP, 8), f32).at[:N_EDGES, :4].set(edge_features)
    pad_idx = jnp.full((EP - N_EDGES,), N_NODES, jnp.int32)
    s_p = jnp.concatenate([senders, pad_idx])
    r_p = jnp.concatenate([receivers, pad_idx])
    zeros_np = jnp.zeros((NP, 2 * LATENT), f32)

    Wee_p = jnp.zeros((8, LATENT), f32).at[:4].set(Wee)

    b1, b2 = blocks
    consts_prep = jnp.stack(
        [ben, gen, bten, b1["bn"], b1["gn"], b1["btn"],
         jnp.zeros_like(ben), jnp.zeros_like(ben)])
    consts_e1 = jnp.stack(
        [bee, gee, btee, b1["be"], b1["ge"], b1["bte"], b2["be"],
         jnp.zeros_like(bee)])
    consts_e2 = jnp.stack(
        [b2["ge"], b2["bte"]] + [jnp.zeros_like(bee)] * 6)
    consts_n1 = jnp.stack(
        [b1["bn"], b1["gn"], b1["btn"]] + [jnp.zeros_like(ben)] * 5)
    consts_n2 = jnp.stack(
        [b2["bn"], b2["gn"], b2["btn"]] + [jnp.zeros_like(ben)] * 5)

    nlat, T1 = _tc_call(
        _prep_body,
        [((NP, LATENT), f32), ((NP, 2 * LATENT), f32)],
        nf_p, Wen, consts_prep, b1["Wsr"])

    gS1, gR1 = _sc_gather(T1, s_p, r_p)
    (p1,) = _tc_edge_call(
        _edge1_body, 1,
        [ef_p, gS1, gR1],
        [Wee_p, b1["We"], b2["We"], consts_e1])
    agg1 = _sc_scatter(p1, r_p, zeros_np)
    nlat1, T2 = _tc_call(
        _node_body,
        [((NP, LATENT), f32), ((NP, 2 * LATENT), f32)],
        nlat, agg1, b1["Wn1"], b1["Wn2"], consts_n1, b2["Wsr"])

    gS2, gR2 = _sc_gather(T2, s_p, r_p)
    (p2,) = _tc_edge_call(
        _edge2_body, 1,
        [p1, gS2, gR2],
        [consts_e2])
    agg2 = _sc_scatter(p2, r_p, zeros_np)
    (out_p,) = _tc_call(
        _node2dec_body,
        [((NP, 128), f32)],
        nlat1, agg2, b2["Wn1"], b2["Wn2"], consts_n2, Wd_p, bd_p)

    return out_p[:N_NODES, :3]


# speedup vs baseline: 1.9018x; 1.9018x over previous
import functools

import jax
import jax.numpy as jnp
from jax import lax
from jax.experimental import pallas as pl
from jax.experimental.pallas import tpu as pltpu
from jax.experimental.pallas import tpu_sc as plsc

N_NODES = 10000
N_EDGES = 320000
D_FEAT = 128
LATENT = 64

NP = 10240
EP = 327680
GW = 128
EBLK = 2048

_EPS = 1e-5


def _collapse(mlp):
    layers = mlp["layers"]
    W, b = layers[0]
    for Wi, bi in layers[1:]:
        W = W @ Wi
        b = b @ Wi + bi
    if mlp["ln"] is not None:
        g, bt = mlp["ln"]
    else:
        g, bt = None, None
    return W, b, g, bt


def _ln(x, g, bt):
    mu = jnp.mean(x, axis=-1, keepdims=True)
    d = x - mu
    var = jnp.mean(d * d, axis=-1, keepdims=True)
    return d * lax.rsqrt(var + _EPS) * g + bt


def _row(c, i):
    return c[i, :][None, :]


def _prep_body(nf_ref, wenc_ref, c_ref, wsr_ref, nlat_ref, t_ref):
    c = c_ref[...]
    x = jnp.dot(nf_ref[...], wenc_ref[...], preferred_element_type=jnp.float32,
                precision=lax.Precision.HIGHEST)
    x = x + _row(c, 0)
    nlat = _ln(x, _row(c, 1), _row(c, 2))
    nlat_ref[...] = nlat
    t_ref[...] = jnp.dot(nlat, wsr_ref[...], preferred_element_type=jnp.float32,
                precision=lax.Precision.HIGHEST)


def _edge1_body(ef_ref, gs_ref, gr_ref, wenc_ref, we1_ref, we2_ref, c_ref,
                p1_ref):
    c = c_ref[...]
    e0 = jnp.dot(ef_ref[...], wenc_ref[...], preferred_element_type=jnp.float32,
                precision=lax.Precision.HIGHEST)
    e0 = _ln(e0 + _row(c, 0), _row(c, 1), _row(c, 2))
    pre = gs_ref[:, 0:LATENT] + gr_ref[:, LATENT:2 * LATENT]
    pre = pre + jnp.dot(e0, we1_ref[...], preferred_element_type=jnp.float32,
                precision=lax.Precision.HIGHEST)
    pre = pre + _row(c, 3)
    ne1 = _ln(pre, _row(c, 4), _row(c, 5))
    el1 = e0 + ne1
    ehat2 = (jnp.dot(el1, we2_ref[...], preferred_element_type=jnp.float32,
                precision=lax.Precision.HIGHEST)
             + _row(c, 6))
    p1_ref[...] = jnp.concatenate([ne1, ehat2], axis=1)


def _edge2_body(p1_ref, gs_ref, gr_ref, c_ref, p2_ref):
    c = c_ref[...]
    pre = (gs_ref[:, 0:LATENT] + gr_ref[:, LATENT:2 * LATENT]
           + p1_ref[:, LATENT:2 * LATENT])
    ne2 = _ln(pre, _row(c, 0), _row(c, 1))
    p2_ref[...] = jnp.concatenate([ne2, jnp.zeros_like(ne2)], axis=1)


def _node_body(nlat_ref, agg_ref, wn1_ref, wn2_ref, c_ref, wsr_ref,
               nlat1_ref, t_ref):
    c = c_ref[...]
    agg = agg_ref[0, :, 0:LATENT] + agg_ref[1, :, 0:LATENT]
    nlat = nlat_ref[...]
    x = jnp.dot(nlat, wn1_ref[...], preferred_element_type=jnp.float32,
                precision=lax.Precision.HIGHEST)
    x = x + jnp.dot(agg, wn2_ref[...], preferred_element_type=jnp.float32,
                precision=lax.Precision.HIGHEST)
    x = x + _row(c, 0)
    nlat1 = nlat + _ln(x, _row(c, 1), _row(c, 2))
    nlat1_ref[...] = nlat1
    t_ref[...] = jnp.dot(nlat1, wsr_ref[...], preferred_element_type=jnp.float32,
                precision=lax.Precision.HIGHEST)


def _node2dec_body(nlat_ref, agg_ref, wn1_ref, wn2_ref, c_ref, wd_ref, bd_ref,
                   out_ref):
    c = c_ref[...]
    agg = agg_ref[0, :, 0:LATENT] + agg_ref[1, :, 0:LATENT]
    nlat = nlat_ref[...]
    x = jnp.dot(nlat, wn1_ref[...], preferred_element_type=jnp.float32,
                precision=lax.Precision.HIGHEST)
    x = x + jnp.dot(agg, wn2_ref[...], preferred_element_type=jnp.float32,
                precision=lax.Precision.HIGHEST)
    x = x + _row(c, 0)
    nlat2 = nlat + _ln(x, _row(c, 1), _row(c, 2))
    out_ref[...] = (
        jnp.dot(nlat2, wd_ref[...], preferred_element_type=jnp.float32,
                precision=lax.Precision.HIGHEST)
        + bd_ref[...])


def _whole(x):
    return pl.BlockSpec(x.shape, lambda *_: (0,) * x.ndim)


def _tc_call(body, outs, *args):
    out_shape = [jax.ShapeDtypeStruct(s, d) for s, d in outs]
    return pl.pallas_call(
        body,
        grid=(1,),
        in_specs=[_whole(a) for a in args],
        out_specs=[pl.BlockSpec(s, lambda *_: (0,) * len(s)) for s, _ in outs],
        out_shape=out_shape,
    )(*args)


def _tc_edge_call(body, n_out, row_args, const_args):
    def rspec(a):
        return pl.BlockSpec((EBLK, a.shape[1]), lambda i: (i, 0))

    in_specs = ([rspec(a) for a in row_args]
                + [_whole(a) for a in const_args])
    out_specs = [pl.BlockSpec((EBLK, 2 * LATENT), lambda i: (i, 0))
                 for _ in range(n_out)]
    out_shape = [jax.ShapeDtypeStruct((EP, 2 * LATENT), jnp.float32)
                 for _ in range(n_out)]
    res = pl.pallas_call(
        body,
        grid=(EP // EBLK,),
        in_specs=in_specs,
        out_specs=out_specs,
        out_shape=out_shape,
    )(*row_args, *const_args)
    return res


def _sc_mesh():
    return plsc.VectorSubcoreMesh(core_axis_name="c", subcore_axis_name="s")


def _sc_gather(tab, s_idx, r_idx):
    GW2 = 2 * GW
    s2 = s_idx.reshape(EP // GW, GW)
    r2 = r_idx.reshape(EP // GW, GW)
    out_type = (jax.ShapeDtypeStruct((EP, 2 * LATENT), jnp.float32),
                jax.ShapeDtypeStruct((EP, 2 * LATENT), jnp.float32))

    @functools.partial(
        pl.kernel, out_type=out_type, mesh=_sc_mesh(),
        scratch_types=[pltpu.SemaphoreType.DMA] * 4)
    def k(t_hbm, s_hbm, r_hbm, os_hbm, or_hbm, s0, s1, s2_, s3):
        def body(s_vmem, r_vmem, os_vmem, or_vmem):
            c0 = pltpu.async_copy(t_hbm.at[s_vmem.at[0]],
                                  os_vmem.at[pl.ds(0, GW)], s0)
            c1 = pltpu.async_copy(t_hbm.at[r_vmem.at[0]],
                                  or_vmem.at[pl.ds(0, GW)], s1)
            c2 = pltpu.async_copy(t_hbm.at[s_vmem.at[1]],
                                  os_vmem.at[pl.ds(GW, GW)], s2_)
            c3 = pltpu.async_copy(t_hbm.at[r_vmem.at[1]],
                                  or_vmem.at[pl.ds(GW, GW)], s3)
            c0.wait()
            c1.wait()
            c2.wait()
            c3.wait()

        obspec = pl.BlockSpec((GW2, 2 * LATENT), lambda i: (i, 0),
                              pipeline_mode=pl.Buffered(buffer_count=1))
        pltpu.emit_pipeline(
            body,
            grid=(EP // GW2,),
            in_specs=[pl.BlockSpec((2, GW), lambda i: (i, 0)),
                      pl.BlockSpec((2, GW), lambda i: (i, 0))],
            out_specs=[obspec, obspec],
            core_axis_name=("c", "s"),
            dimension_semantics=(pltpu.PARALLEL,),
        )(s_hbm, r_hbm, os_hbm, or_hbm)

    return k(tab, s2, r2)


def _sc_scatter(ne, r_idx, zeros_np):
    W = ne.shape[1]
    r2 = r_idx.reshape(1, EP)
    rows = NP // 16
    out_type = jax.ShapeDtypeStruct((2, NP, W), jnp.float32)

    @functools.partial(
        pl.kernel, out_type=out_type, mesh=_sc_mesh(),
        scratch_types=[pltpu.VMEM_SHARED((NP, W), jnp.float32)])
    def k(ne_hbm, r_hbm, z_hbm, out_hbm, acc):
        cid = lax.axis_index("c")
        sid = lax.axis_index("s")
        base = pl.multiple_of(sid * rows, 8)
        pltpu.sync_copy(z_hbm.at[pl.ds(base, rows)],
                        acc.at[pl.ds(base, rows)])
        plsc.subcore_barrier()

        def body(ne_vmem, i_vmem):
            pltpu.sync_copy(ne_vmem, acc.at[i_vmem.at[0]], add=True)

        pltpu.emit_pipeline(
            body,
            grid=(EP // GW,),
            in_specs=[pl.BlockSpec((GW, W), lambda i: (i, 0)),
                      pl.BlockSpec((1, GW), lambda i: (0, i))],
            out_specs=[],
            core_axis_name=("c", "s"),
            dimension_semantics=(pltpu.PARALLEL,),
        )(ne_hbm, r_hbm)

        plsc.subcore_barrier()
        pltpu.sync_copy(acc.at[pl.ds(base, rows)],
                        out_hbm.at[cid, pl.ds(base, rows)])

    return k(ne, r2, zeros_np)


def _sc_gather_spmem(tab, s_idx, r_idx):
    s2 = s_idx.reshape(1, EP)
    r2 = r_idx.reshape(1, EP)
    rows = NP // 16
    out_type = (jax.ShapeDtypeStruct((EP, 2 * LATENT), jnp.float32),
                jax.ShapeDtypeStruct((EP, 2 * LATENT), jnp.float32))

    @functools.partial(
        pl.kernel, out_type=out_type, mesh=_sc_mesh(),
        scratch_types=[pltpu.VMEM_SHARED((NP, 2 * LATENT), jnp.float32),
                       pltpu.SemaphoreType.DMA, pltpu.SemaphoreType.DMA])
    def k(t_hbm, s_hbm, r_hbm, os_hbm, or_hbm, stab, sem_s, sem_r):
        sid = lax.axis_index("s")
        base = pl.multiple_of(sid * rows, 8)
        pltpu.sync_copy(t_hbm.at[pl.ds(base, rows)],
                        stab.at[pl.ds(base, rows)])
        plsc.subcore_barrier()

        def body(s_vmem, r_vmem, os_vmem, or_vmem):
            cs = pltpu.async_copy(stab.at[s_vmem.at[0]], os_vmem, sem_s)
            cr = pltpu.async_copy(stab.at[r_vmem.at[0]], or_vmem, sem_r)
            cs.wait()
            cr.wait()

        ob = pl.BlockSpec((GW, 2 * LATENT), lambda i: (i, 0),
                          pipeline_mode=pl.Buffered(buffer_count=1))
        pltpu.emit_pipeline(
            body,
            grid=(EP // GW,),
            in_specs=[pl.BlockSpec((1, GW), lambda i: (0, i)),
                      pl.BlockSpec((1, GW), lambda i: (0, i))],
            out_specs=[ob, ob],
            core_axis_name=("c", "s"),
            dimension_semantics=(pltpu.PARALLEL,),
        )(s_hbm, r_hbm, os_hbm, or_hbm)

    return k(tab, s2, r2)


def kernel(node_features, edge_features, senders, receivers, params):
    f32 = jnp.float32

    Wen, ben, gen, bten = _collapse(params["enc_node"])
    Wee, bee, gee, btee = _collapse(params["enc_edge"])
    blocks = []
    for blk in params["blocks"]:
        We, be, ge, bte = _collapse(blk["edge"])
        Wn, bn, gn, btn = _collapse(blk["node"])
        Wsr = jnp.concatenate([We[0:64], We[64:128]], axis=1)
        blocks.append(dict(
            Wsr=Wsr, We=We[128:192], be=be, ge=ge, bte=bte,
            Wn1=Wn[0:64], Wn2=Wn[64:128], bn=bn, gn=gn, btn=btn))
    Wd, bd, _, _ = _collapse(params["dec"])
    Wd_p = jnp.zeros((LATENT, 128), f32).at[:, :3].set(Wd)
    bd_p = jnp.zeros((1, 128), f32).at[0, :3].set(bd)

    nf_p = jnp.zeros((NP, D_FEAT), f32).at[:N_NODES].set(node_features)
    ef_p = jnp.zeros((EP, 8), f32).at[:N_EDGES, :4].set(edge_features)
    pad_idx = jnp.full((EP - N_EDGES,), N_NODES, jnp.int32)
    s_p = jnp.concatenate([senders, pad_idx])
    r_p = jnp.concatenate([receivers, pad_idx])
    zeros_np = jnp.zeros((NP, 2 * LATENT), f32)

    Wee_p = jnp.zeros((8, LATENT), f32).at[:4].set(Wee)

    b1, b2 = blocks
    consts_prep = jnp.stack(
        [ben, gen, bten, b1["bn"], b1["gn"], b1["btn"],
         jnp.zeros_like(ben), jnp.zeros_like(ben)])
    consts_e1 = jnp.stack(
        [bee, gee, btee, b1["be"], b1["ge"], b1["bte"], b2["be"],
         jnp.zeros_like(bee)])
    consts_e2 = jnp.stack(
        [b2["ge"], b2["bte"]] + [jnp.zeros_like(bee)] * 6)
    consts_n1 = jnp.stack(
        [b1["bn"], b1["gn"], b1["btn"]] + [jnp.zeros_like(ben)] * 5)
    consts_n2 = jnp.stack(
        [b2["bn"], b2["gn"], b2["btn"]] + [jnp.zeros_like(ben)] * 5)

    nlat, T1 = _tc_call(
        _prep_body,
        [((NP, LATENT), f32), ((NP, 2 * LATENT), f32)],
        nf_p, Wen, consts_prep, b1["Wsr"])

    gS1, gR1 = _sc_gather_spmem(T1, s_p, r_p)
    (p1,) = _tc_edge_call(
        _edge1_body, 1,
        [ef_p, gS1, gR1],
        [Wee_p, b1["We"], b2["We"], consts_e1])
    agg1 = _sc_scatter(p1, r_p, zeros_np)
    nlat1, T2 = _tc_call(
        _node_body,
        [((NP, LATENT), f32), ((NP, 2 * LATENT), f32)],
        nlat, agg1, b1["Wn1"], b1["Wn2"], consts_n1, b2["Wsr"])

    gS2, gR2 = _sc_gather_spmem(T2, s_p, r_p)
    (p2,) = _tc_edge_call(
        _edge2_body, 1,
        [p1, gS2, gR2],
        [consts_e2])
    agg2 = _sc_scatter(p2, r_p, zeros_np)
    (out_p,) = _tc_call(
        _node2dec_body,
        [((NP, 128), f32)],
        nlat1, agg2, b2["Wn1"], b2["Wn2"], consts_n2, Wd_p, bd_p)

    return out_p[:N_NODES, :3]
